# baseline (speedup 1.0000x reference)
import math

import jax
import jax.numpy as jnp
import numpy as np
from jax.experimental import pallas as pl
from jax.experimental.pallas import tpu as pltpu


_D_MODEL = 1024
_HALF = _D_MODEL // 2
_T = 32
_BLK = 2048
_SEQ = 4096
_BATCH = 4


def _factor_tables():
    scale = -np.log(10000.0) / (_HALF - 1)
    div = np.exp(np.arange(_HALF) * scale)
    alpha = (np.arange(_SEQ // _T) * _T)[:, None] * div
    beta = np.arange(_T)[:, None] * div
    return (
        jnp.asarray(np.sin(alpha), dtype=jnp.float32),
        jnp.asarray(np.cos(alpha), dtype=jnp.float32),
        jnp.asarray(np.sin(beta), dtype=jnp.float32),
        jnp.asarray(np.cos(beta), dtype=jnp.float32),
    )


def _pe_add_body(x_ref, sa_ref, ca_ref, sb_ref, cb_ref, o_ref, psin, pcos):
    i = pl.program_id(0)
    q_grp = _BLK // _T

    @pl.when(jax.lax.rem(i, _BATCH) == 0)
    def _compute_pe():
        sa = sa_ref[...].reshape(q_grp, 1, _HALF)
        ca = ca_ref[...].reshape(q_grp, 1, _HALF)
        sb = sb_ref[...].reshape(1, _T, _HALF)
        cb = cb_ref[...].reshape(1, _T, _HALF)
        psin[...] = (sa * cb + ca * sb).reshape(_BLK, _HALF)
        pcos[...] = (ca * cb - sa * sb).reshape(_BLK, _HALF)

    o_ref[:, :_HALF] = x_ref[:, :_HALF] + psin[...]
    o_ref[:, _HALF:] = x_ref[:, _HALF:] + pcos[...]


def kernel(inputs):
    batch, seq_len, d_model = inputs.shape
    assert (batch, seq_len, d_model) == (_BATCH, _SEQ, _D_MODEL)
    rows = batch * seq_len
    x = inputs.reshape(rows, d_model)
    sa, ca, sb, cb = _factor_tables()
    q_grp = _BLK // _T
    hblocks = seq_len // _BLK

    out = pl.pallas_call(
        _pe_add_body,
        grid=(rows // _BLK,),
        in_specs=[
            pl.BlockSpec(
                (_BLK, d_model),
                lambda i: ((i % _BATCH) * hblocks + i // _BATCH, 0),
            ),
            pl.BlockSpec((q_grp, _HALF), lambda i: (i // _BATCH, 0)),
            pl.BlockSpec((q_grp, _HALF), lambda i: (i // _BATCH, 0)),
            pl.BlockSpec((_T, _HALF), lambda i: (0, 0)),
            pl.BlockSpec((_T, _HALF), lambda i: (0, 0)),
        ],
        out_specs=pl.BlockSpec(
            (_BLK, d_model),
            lambda i: ((i % _BATCH) * hblocks + i // _BATCH, 0),
        ),
        out_shape=jax.ShapeDtypeStruct((rows, d_model), inputs.dtype),
        scratch_shapes=[
            pltpu.VMEM((_BLK, _HALF), jnp.float32),
            pltpu.VMEM((_BLK, _HALF), jnp.float32),
        ],
    )(x, sa, ca, sb, cb)
    return out.reshape(batch, seq_len, d_model)


# speedup vs baseline: 1.0124x; 1.0124x over previous
import math

import jax
import jax.numpy as jnp
import numpy as np
from jax.experimental import pallas as pl


_D_MODEL = 1024
_HALF = _D_MODEL // 2
_T = 32
_BLK = 512
_SEQ = 4096


def _factor_tables():
    scale = -np.log(10000.0) / (_HALF - 1)
    div = np.exp(np.arange(_HALF) * scale)
    alpha = (np.arange(_SEQ // _T) * _T)[:, None] * div
    beta = np.arange(_T)[:, None] * div
    return (
        jnp.asarray(np.sin(alpha), dtype=jnp.float32),
        jnp.asarray(np.cos(alpha), dtype=jnp.float32),
        jnp.asarray(np.sin(beta), dtype=jnp.float32),
        jnp.asarray(np.cos(beta), dtype=jnp.float32),
    )


def _pe_add_body(x_ref, sa_ref, ca_ref, sb_ref, cb_ref, o_ref):
    q_grp = _BLK // _T
    sa = sa_ref[...].reshape(q_grp, 1, _HALF)
    ca = ca_ref[...].reshape(q_grp, 1, _HALF)
    sb = sb_ref[...].reshape(1, _T, _HALF)
    cb = cb_ref[...].reshape(1, _T, _HALF)
    pe_sin = (sa * cb + ca * sb).reshape(_BLK, _HALF)
    pe_cos = (ca * cb - sa * sb).reshape(_BLK, _HALF)
    o_ref[:, :, :_HALF] = x_ref[:, :, :_HALF] + pe_sin[None]
    o_ref[:, :, _HALF:] = x_ref[:, :, _HALF:] + pe_cos[None]


def kernel(inputs):
    batch, seq_len, d_model = inputs.shape
    assert d_model == _D_MODEL and seq_len == _SEQ
    sa, ca, sb, cb = _factor_tables()
    q_grp = _BLK // _T
    out = pl.pallas_call(
        _pe_add_body,
        grid=(seq_len // _BLK,),
        in_specs=[
            pl.BlockSpec((batch, _BLK, d_model), lambda i: (0, i, 0)),
            pl.BlockSpec((q_grp, _HALF), lambda i: (i, 0)),
            pl.BlockSpec((q_grp, _HALF), lambda i: (i, 0)),
            pl.BlockSpec((_T, _HALF), lambda i: (0, 0)),
            pl.BlockSpec((_T, _HALF), lambda i: (0, 0)),
        ],
        out_specs=pl.BlockSpec((batch, _BLK, d_model), lambda i: (0, i, 0)),
        out_shape=jax.ShapeDtypeStruct(inputs.shape, inputs.dtype),
    )(inputs, sa, ca, sb, cb)
    return out


# speedup vs baseline: 1.0164x; 1.0040x over previous
import jax
import jax.numpy as jnp
import numpy as np
from jax.experimental import pallas as pl


_D_MODEL = 1024
_HALF = _D_MODEL // 2
_T = 32
_BLK = 512
_SEQ = 4096


def _factor_tables():
    scale = -np.log(10000.0) / (_HALF - 1)
    div = np.exp(np.arange(_HALF) * scale)
    alpha = (np.arange(_SEQ // _T) * _T)[:, None] * div
    beta = np.arange(_T)[:, None] * div
    return (
        jnp.asarray(np.sin(alpha), dtype=jnp.float32),
        jnp.asarray(np.cos(alpha), dtype=jnp.float32),
        jnp.asarray(np.sin(beta), dtype=jnp.float32),
        jnp.asarray(np.cos(beta), dtype=jnp.float32),
    )


def _pe_add_body(x_ref, sa_ref, ca_ref, sb_ref, cb_ref, o_ref):
    q_grp = _BLK // _T
    sa = sa_ref[...].reshape(q_grp, 1, _HALF)
    ca = ca_ref[...].reshape(q_grp, 1, _HALF)
    sb = sb_ref[...].reshape(1, _T, _HALF)
    cb = cb_ref[...].reshape(1, _T, _HALF)
    pe_sin = (sa * cb + ca * sb).reshape(_BLK, _HALF)
    pe_cos = (ca * cb - sa * sb).reshape(_BLK, _HALF)
    o_ref[:, :, :_HALF] = x_ref[:, :, :_HALF] + pe_sin[None]
    o_ref[:, :, _HALF:] = x_ref[:, :, _HALF:] + pe_cos[None]


def kernel(inputs):
    batch, seq_len, d_model = inputs.shape
    assert d_model == _D_MODEL and seq_len == _SEQ
    sa, ca, sb, cb = _factor_tables()
    q_grp = _BLK // _T
    out = pl.pallas_call(
        _pe_add_body,
        grid=(seq_len // _BLK,),
        in_specs=[
            pl.BlockSpec((batch, _BLK, d_model), lambda i: (0, i, 0)),
            pl.BlockSpec((q_grp, _HALF), lambda i: (i, 0)),
            pl.BlockSpec((q_grp, _HALF), lambda i: (i, 0)),
            pl.BlockSpec((_T, _HALF), lambda i: (0, 0)),
            pl.BlockSpec((_T, _HALF), lambda i: (0, 0)),
        ],
        out_specs=pl.BlockSpec((batch, _BLK, d_model), lambda i: (0, i, 0)),
        out_shape=jax.ShapeDtypeStruct(inputs.shape, inputs.dtype),
    )(inputs, sa, ca, sb, cb)
    return out
